# baseline (speedup 1.0000x reference)
import functools

import jax
import jax.numpy as jnp
import numpy as np
from jax import lax
from jax.experimental import pallas as pl
from jax.experimental.pallas import tpu as pltpu

N_ROWS = 16384
N_CLASSES = 1000
NBINS = 10
ROW_BLOCK = 2048

_BOUNDS = np.array(
    [0x00000000, 0x3DCCCCCD, 0x3E4CCCCD, 0x3E99999A, 0x3ECCCCCD, 0x3F000000,
     0x3F19999A, 0x3F333333, 0x3F4CCCCD, 0x3F666667, 0x3F800000],
    dtype=np.uint32,
).view(np.float32)

_LO = np.full((1, 128), 2.0, np.float32)
_HI = np.full((1, 128), 3.0, np.float32)
_LO[0, :NBINS] = _BOUNDS[:NBINS]
_HI[0, :NBINS] = _BOUNDS[1:]


def _ece_tc_kernel(x_ref, t_ref, out_ref, hist_ref):
    i = pl.program_id(0)
    nsteps = pl.num_programs(0)

    x = x_ref[...]
    t = t_ref[...]
    R, C = x.shape

    col = lax.broadcasted_iota(jnp.int32, (R, C), 1)
    m = jnp.max(x, axis=1, keepdims=True)
    e = jnp.exp(x - m)
    s = jnp.sum(e, axis=1, keepdims=True)
    te = jnp.sum(jnp.where(col == t, e, 0.0), axis=1, keepdims=True)
    p = te / s
    am = jnp.min(jnp.where(x == m, col, C), axis=1, keepdims=True)
    correct = (am == t).astype(jnp.float32)

    lane = lax.broadcasted_iota(jnp.int32, (1, 128), 1)
    lo = jnp.full((1, 128), 2.0, jnp.float32)
    hi = jnp.full((1, 128), 3.0, jnp.float32)
    for k in range(NBINS):
        lo = jnp.where(lane == k, float(_BOUNDS[k]), lo)
        hi = jnp.where(lane == k, float(_BOUNDS[k + 1]), hi)
    onehot = ((p > lo) & (p <= hi)).astype(jnp.float32)
    cnt = jnp.sum(onehot, axis=0, keepdims=True)
    sp = jnp.sum(onehot * p, axis=0, keepdims=True)
    sc = jnp.sum(onehot * correct, axis=0, keepdims=True)

    @pl.when(i == 0)
    def _():
        hist_ref[...] = jnp.zeros_like(hist_ref)

    hist_ref[0:1, :] += cnt
    hist_ref[1:2, :] += sp
    hist_ref[2:3, :] += sc

    @pl.when(i == nsteps - 1)
    def _():
        cntv = hist_ref[0:1, :]
        spv = hist_ref[1:2, :]
        scv = hist_ref[2:3, :]
        safe = jnp.maximum(cntv, 1.0)
        term = jnp.where(cntv > 0, cntv * jnp.abs(spv / safe - scv / safe), 0.0)
        total = jnp.sum(cntv, keepdims=True)
        ece = jnp.where(total > 0, jnp.sum(term, keepdims=True) / total, 0.0)
        out_ref[...] = ece


@functools.partial(jax.jit, static_argnames=("interpret",))
def _ece(outputs, targets, interpret=False):
    t2d = targets.astype(jnp.int32).reshape(N_ROWS, 1)
    grid = N_ROWS // ROW_BLOCK
    out = pl.pallas_call(
        _ece_tc_kernel,
        grid=(grid,),
        in_specs=[
            pl.BlockSpec((ROW_BLOCK, N_CLASSES), lambda i: (i, 0)),
            pl.BlockSpec((ROW_BLOCK, 1), lambda i: (i, 0)),
        ],
        out_specs=pl.BlockSpec((1, 1), lambda i: (0, 0)),
        out_shape=jax.ShapeDtypeStruct((1, 1), jnp.float32),
        scratch_shapes=[pltpu.VMEM((8, 128), jnp.float32)],
        interpret=interpret,
    )(outputs, t2d)
    return out.reshape(())


def kernel(outputs, targets):
    return _ece(outputs, targets)
